# baseline (speedup 1.0000x reference)
import jax
import jax.numpy as jnp
from jax.experimental import pallas as pl

RB = 16


def _argmin_kernel(xb_ref, et_ref, idx_ref):
    cols = []
    for c in range(4):
        xc = xb_ref[0, :, c, :]

        def dk(k):
            d = xc[:, k:k + 1] - et_ref[c, k]
            return d * d

        d2 = None
        for j in range(8):
            b = 8 * j
            p0 = dk(b + 0) + dk(b + 4)
            p2 = dk(b + 2) + dk(b + 6)
            p1 = dk(b + 1) + dk(b + 5)
            p3 = dk(b + 3) + dk(b + 7)
            v = (p0 + p2) + (p1 + p3)
            d2 = v if d2 is None else d2 + v
        s = jnp.sqrt(d2)
        mn = jnp.min(s, axis=1, keepdims=True)
        iota = jax.lax.broadcasted_iota(jnp.int32, s.shape, 1)
        idx = jnp.min(jnp.where(s == mn, iota, 1024), axis=1)
        cols.append(idx[:, None])
    idx_ref[...] = jnp.concatenate(cols, axis=1)


def _assemble_kernel(idx_ref, emb_ref, x_ref, out0_ref, out1_ref, hist_ref,
                     ent_ref):
    hist = jnp.zeros((1, 1024), dtype=jnp.float32)
    for c in range(4):
        idxc = idx_ref[:, c:c + 1]
        iota = jax.lax.broadcasted_iota(jnp.int32, (2048, 1024), 1)
        onehot = (iota == idxc).astype(jnp.float32)
        g = jax.lax.dot(onehot, emb_ref[c],
                        precision=jax.lax.Precision.HIGHEST)
        hist = hist + jnp.sum(onehot, axis=0, keepdims=True)
        x = x_ref[c]
        out0_ref[c] = (g - x) + x
        t = x - g
        out1_ref[c] = jnp.sum(t * t, axis=1, keepdims=True)
    hist_ref[...] = hist
    p = hist * jnp.float32(1.0 / 2048.0)
    pos = hist > 0
    safe = jnp.where(pos, p, jnp.float32(1.0))
    ent = -jnp.sum(jnp.where(pos, p * jnp.log(safe), jnp.float32(0.0)))
    ent_ref[...] = ent.reshape(1, 1)


def kernel(x0, embedding0):
    x2 = x0.reshape(2048, 4, 64)
    xb = x2.reshape(2048 // RB, RB, 4, 64)
    et = embedding0.transpose(0, 2, 1)
    etb = jnp.broadcast_to(et[:, :, None, :], (4, 64, RB, 1024))

    idx = pl.pallas_call(
        _argmin_kernel,
        grid=(2048 // RB,),
        in_specs=[
            pl.BlockSpec((1, RB, 4, 64), lambda i: (i, 0, 0, 0)),
            pl.BlockSpec((4, 64, RB, 1024), lambda i: (0, 0, 0, 0)),
        ],
        out_specs=pl.BlockSpec((RB, 4), lambda i: (i, 0)),
        out_shape=jax.ShapeDtypeStruct((2048, 4), jnp.int32),
    )(xb, etb)

    xc = x2.transpose(1, 0, 2)
    out0c, out1c, hist, ent = pl.pallas_call(
        _assemble_kernel,
        grid=(1,),
        in_specs=[
            pl.BlockSpec((2048, 4), lambda i: (0, 0)),
            pl.BlockSpec((4, 1024, 64), lambda i: (0, 0, 0)),
            pl.BlockSpec((4, 2048, 64), lambda i: (0, 0, 0)),
        ],
        out_specs=[
            pl.BlockSpec((4, 2048, 64), lambda i: (0, 0, 0)),
            pl.BlockSpec((4, 2048, 1), lambda i: (0, 0, 0)),
            pl.BlockSpec((1, 1024), lambda i: (0, 0)),
            pl.BlockSpec((1, 1), lambda i: (0, 0)),
        ],
        out_shape=[
            jax.ShapeDtypeStruct((4, 2048, 64), jnp.float32),
            jax.ShapeDtypeStruct((4, 2048, 1), jnp.float32),
            jax.ShapeDtypeStruct((1, 1024), jnp.float32),
            jax.ShapeDtypeStruct((1, 1), jnp.float32),
        ],
    )(idx, embedding0, xc)

    out0 = out0c.transpose(1, 0, 2).reshape(4, 512, 4, 64)
    out1 = out1c.reshape(4, 2048).transpose(1, 0).reshape(4, 512, 4)
    entropy = ent.reshape(())
    return (out0, out1, out1, entropy)


# speedup vs baseline: 6.0777x; 6.0777x over previous
import jax
import jax.numpy as jnp
from jax.experimental import pallas as pl

RB = 16


def _argmin_kernel(xb_ref, et_ref, idx_ref):
    cols = []
    for c in range(4):
        xc = xb_ref[0, c]

        def dk(k):
            d = xc[:, k:k + 1] - et_ref[c, k]
            return d * d

        d2 = None
        for j in range(8):
            b = 8 * j
            p0 = dk(b + 0) + dk(b + 4)
            p2 = dk(b + 2) + dk(b + 6)
            p1 = dk(b + 1) + dk(b + 5)
            p3 = dk(b + 3) + dk(b + 7)
            v = (p0 + p2) + (p1 + p3)
            d2 = v if d2 is None else d2 + v
        s = jnp.sqrt(d2)
        mn = jnp.min(s, axis=1, keepdims=True)
        iota = jax.lax.broadcasted_iota(jnp.int32, s.shape, 1)
        idx = jnp.min(jnp.where(s == mn, iota, 1024), axis=1)
        cols.append(idx[:, None])
    idx_ref[...] = jnp.concatenate(cols, axis=1)


def _assemble_kernel(idx_ref, emb_ref, x_ref, out0_ref, out1_ref, hist_ref,
                     ent_ref):
    hist = jnp.zeros((1, 1024), dtype=jnp.float32)
    for c in range(4):
        idxc = idx_ref[:, c:c + 1]
        iota = jax.lax.broadcasted_iota(jnp.int32, (2048, 1024), 1)
        onehot = (iota == idxc).astype(jnp.float32)
        g = jax.lax.dot(onehot, emb_ref[c],
                        precision=jax.lax.Precision.HIGHEST)
        hist = hist + jnp.sum(onehot, axis=0, keepdims=True)
        x = x_ref[c]
        out0_ref[c] = (g - x) + x
        t = x - g
        out1_ref[c] = jnp.sum(t * t, axis=1, keepdims=True)
    hist_ref[...] = hist
    p = hist * jnp.float32(1.0 / 2048.0)
    pos = hist > 0
    safe = jnp.where(pos, p, jnp.float32(1.0))
    ent = -jnp.sum(jnp.where(pos, p * jnp.log(safe), jnp.float32(0.0)))
    ent_ref[...] = ent.reshape(1, 1)


def kernel(x0, embedding0):
    x2 = x0.reshape(2048, 4, 64)
    xb = x2.reshape(2048 // RB, RB, 4, 64).transpose(0, 2, 1, 3)
    et = embedding0.transpose(0, 2, 1)
    etb = jnp.broadcast_to(et[:, :, None, :], (4, 64, RB, 1024))

    idx = pl.pallas_call(
        _argmin_kernel,
        grid=(2048 // RB,),
        in_specs=[
            pl.BlockSpec((1, 4, RB, 64), lambda i: (i, 0, 0, 0)),
            pl.BlockSpec((4, 64, RB, 1024), lambda i: (0, 0, 0, 0)),
        ],
        out_specs=pl.BlockSpec((RB, 4), lambda i: (i, 0)),
        out_shape=jax.ShapeDtypeStruct((2048, 4), jnp.int32),
    )(xb, etb)

    xc = x2.transpose(1, 0, 2)
    out0c, out1c, hist, ent = pl.pallas_call(
        _assemble_kernel,
        grid=(1,),
        in_specs=[
            pl.BlockSpec((2048, 4), lambda i: (0, 0)),
            pl.BlockSpec((4, 1024, 64), lambda i: (0, 0, 0)),
            pl.BlockSpec((4, 2048, 64), lambda i: (0, 0, 0)),
        ],
        out_specs=[
            pl.BlockSpec((4, 2048, 64), lambda i: (0, 0, 0)),
            pl.BlockSpec((4, 2048, 1), lambda i: (0, 0, 0)),
            pl.BlockSpec((1, 1024), lambda i: (0, 0)),
            pl.BlockSpec((1, 1), lambda i: (0, 0)),
        ],
        out_shape=[
            jax.ShapeDtypeStruct((4, 2048, 64), jnp.float32),
            jax.ShapeDtypeStruct((4, 2048, 1), jnp.float32),
            jax.ShapeDtypeStruct((1, 1024), jnp.float32),
            jax.ShapeDtypeStruct((1, 1), jnp.float32),
        ],
    )(idx, embedding0, xc)

    out0 = out0c.transpose(1, 0, 2).reshape(4, 512, 4, 64)
    out1 = out1c.reshape(4, 2048).transpose(1, 0).reshape(4, 512, 4)
    entropy = ent.reshape(())
    return (out0, out1, out1, entropy)
